# baseline (speedup 1.0000x reference)
import functools

import jax
import jax.numpy as jnp
from jax.experimental import pallas as pl
from jax.experimental.pallas import tpu as pltpu

N = 10000
E = 160000
F_IN = 128
D_EDGE = 16
HID = 64
HEADS = 4
HD = HEADS * HID

NP_PAD = 10240
NODE_BLK = 1024
EDGE_BLK = 4000


def _node_encode_body(x_ref, ws_ref, bs_ref,
                      t1w0, t1b0, t2w0, t2b0,
                      t1w1, t1b1, t2w1, t2b1,
                      t1w2, t1b2, t2w2, t2b2, o_ref):
    xb = x_ref[...]
    h = jnp.dot(xb, ws_ref[...], preferred_element_type=jnp.float32) + bs_ref[...]
    tf = xb[:, 0:1]
    scales = (1.0, 5.0, 20.0)
    gates = ((t1w0, t1b0, t2w0, t2b0), (t1w1, t1b1, t2w1, t2b1),
             (t1w2, t1b2, t2w2, t2b2))
    for i in range(3):
        t1w, t1b, t2w, t2b = gates[i]
        nt = tf / scales[i]
        a = jax.nn.relu(nt * t1w[...] + t1b[...])
        attn = jax.nn.sigmoid(
            jnp.dot(a, t2w[...], preferred_element_type=jnp.float32) + t2b[...])
        h = h * attn
    o_ref[...] = h


def _node_encode(x, ws, bs, tw1, tb1, tw2, tb2):
    xp = jnp.pad(x, ((0, NP_PAD - N), (0, 0)))
    grid = (NP_PAD // NODE_BLK,)
    gate_args = []
    for i in range(3):
        gate_args += [tw1[i].reshape(1, HID // 4), tb1[i].reshape(1, HID // 4),
                      tw2[i].reshape(HID // 4, 1), tb2[i].reshape(1, 1)]
    small = lambda a: pl.BlockSpec(a.shape, lambda i: tuple(0 for _ in a.shape))
    out = pl.pallas_call(
        _node_encode_body,
        grid=grid,
        in_specs=[pl.BlockSpec((NODE_BLK, F_IN), lambda i: (i, 0)),
                  small(ws), small(bs.reshape(1, HID))]
                 + [small(a) for a in gate_args],
        out_specs=pl.BlockSpec((NODE_BLK, HID), lambda i: (i, 0)),
        out_shape=jax.ShapeDtypeStruct((NP_PAD, HID), jnp.float32),
    )(xp, ws, bs.reshape(1, HID), *gate_args)
    return out[:N]


def _edge_encode_body(ea_ref, ew1_ref, eb1_ref, ew2_ref, eb2_ref,
                      sw1_ref, sb1_ref, sw2_ref, sb2_ref, o_ref):
    ea = ea_ref[...]
    ew = jax.nn.sigmoid(
        jnp.dot(jax.nn.relu(jnp.dot(ea, ew1_ref[...], preferred_element_type=jnp.float32)
                            + eb1_ref[...]),
                ew2_ref[...], preferred_element_type=jnp.float32) + eb2_ref[...])
    sin_a = ea[:, D_EDGE - 2:D_EDGE - 1]
    cos_a = ea[:, D_EDGE - 1:D_EDGE]
    dist = 1.0 - ea[:, 0:1]
    sf = jnp.concatenate([sin_a, cos_a, dist], axis=1)
    se = jnp.tanh(
        jnp.dot(jax.nn.relu(jnp.dot(sf, sw1_ref[...], preferred_element_type=jnp.float32)
                            + sb1_ref[...]),
                sw2_ref[...], preferred_element_type=jnp.float32) + sb2_ref[...])
    o_ref[...] = ew * se


def _edge_encode(edge_attr, ew1, eb1, ew2, eb2, sw1, sb1, sw2, sb2):
    grid = (E // EDGE_BLK,)
    args = (ew1, eb1.reshape(1, HID), ew2, eb2.reshape(1, HID),
            sw1, sb1.reshape(1, HID // 2), sw2, sb2.reshape(1, HID))
    small = lambda a: pl.BlockSpec(a.shape, lambda i: tuple(0 for _ in a.shape))
    out = pl.pallas_call(
        _edge_encode_body,
        grid=grid,
        in_specs=[pl.BlockSpec((EDGE_BLK, D_EDGE), lambda i: (i, 0))]
                 + [small(a) for a in args],
        out_specs=pl.BlockSpec((EDGE_BLK, HID), lambda i: (i, 0)),
        out_shape=jax.ShapeDtypeStruct((E, HID), jnp.float32),
    )(edge_attr, *args)
    return out


def _gatv2(h, src, dst, eattr, Wl, Wr, We, att, bias):
    n = h.shape[0]
    ones = jnp.ones((eattr.shape[0],), jnp.float32)
    cnt = jax.ops.segment_sum(ones, dst, num_segments=n)
    loop_attr = jax.ops.segment_sum(eattr, dst, num_segments=n) / jnp.maximum(cnt, 1.0)[:, None]
    ar = jnp.arange(n, dtype=src.dtype)
    src2 = jnp.concatenate([src, ar])
    dst2 = jnp.concatenate([dst, ar])
    ea2 = jnp.concatenate([eattr, loop_attr], axis=0)
    xl = (h @ Wl).reshape(n, HEADS, HID)
    xr = (h @ Wr).reshape(n, HEADS, HID)
    ee = (ea2 @ We).reshape(-1, HEADS, HID)
    m = jax.nn.leaky_relu(xl[src2] + xr[dst2] + ee, negative_slope=0.2)
    logits = jnp.einsum('ehc,hc->eh', m, att)
    lmax = jax.lax.stop_gradient(jax.ops.segment_max(logits, dst2, num_segments=n))
    lmax = jnp.where(jnp.isfinite(lmax), lmax, 0.0)
    ex = jnp.exp(logits - lmax[dst2])
    den = jax.ops.segment_sum(ex, dst2, num_segments=n)
    alpha = ex / (den[dst2] + 1e-16)
    out = jax.ops.segment_sum(xl[src2] * alpha[:, :, None], dst2, num_segments=n)
    return out.mean(axis=1) + bias


def kernel(x, edge_index, edge_attr, ws, bs, tw1, tb1, tw2, tb2, sw1, sb1, sw2, sb2,
           ew1, eb1, ew2, eb2, Wl, Wr, We, att, gb, bn_g, bn_b,
           hw1, hb1, hw2, hb2, dw1, db1, dw2, db2):
    src, dst = edge_index[0], edge_index[1]
    h = _node_encode(x, ws, bs, tw1, tb1, tw2, tb2)
    edge_weights = _edge_encode(edge_attr, ew1, eb1, ew2, eb2, sw1, sb1, sw2, sb2)
    h = jax.nn.elu(_gatv2(h, src, dst, edge_weights, Wl[0], Wr[0], We[0], att[0], gb[0]))
    for l in range(1, 3):
        h_new = _gatv2(h, src, dst, edge_weights, Wl[l], Wr[l], We[l], att[l], gb[l])
        h = jax.nn.elu(h + h_new)
    h = (h / jnp.sqrt(1.0 + 1e-5)) * bn_g + bn_b
    th = h[0:1]
    hc = jax.nn.relu(th @ hw1 + hb1) @ hw2 + hb2
    dp = jax.nn.relu(th @ dw1 + db1) @ dw2 + db2
    return jnp.concatenate([hc, dp], axis=1)


# speedup vs baseline: 4.0152x; 4.0152x over previous
import functools

import jax
import jax.numpy as jnp
from jax import lax
from jax.experimental import pallas as pl
from jax.experimental.pallas import tpu as pltpu
from jax.experimental.pallas import tpu_sc as plsc

N = 10000
E = 160000
F_IN = 128
D_EDGE = 16
HID = 64
HEADS = 4
HD = HEADS * HID

NP_PAD = 10240
NODE_BLK = 1024
E2 = E + N
E2_PAD = 172032
EDGE_BLK = 2048

NW = 32
GCHUNK = 256


def _small(a):
    return pl.BlockSpec(a.shape, lambda i: tuple(0 for _ in a.shape))


def _node_encode_body(x_ref, ws_ref, bs_ref,
                      t1w0, t1b0, t2w0, t2b0,
                      t1w1, t1b1, t2w1, t2b1,
                      t1w2, t1b2, t2w2, t2b2, o_ref):
    xb = x_ref[...]
    h = jnp.dot(xb, ws_ref[...], preferred_element_type=jnp.float32) + bs_ref[...]
    tf = xb[:, 0:1]
    scales = (1.0, 5.0, 20.0)
    gates = ((t1w0, t1b0, t2w0, t2b0), (t1w1, t1b1, t2w1, t2b1),
             (t1w2, t1b2, t2w2, t2b2))
    for i in range(3):
        t1w, t1b, t2w, t2b = gates[i]
        nt = tf / scales[i]
        a = jax.nn.relu(nt * t1w[...] + t1b[...])
        attn = jax.nn.sigmoid(
            jnp.dot(a, t2w[...], preferred_element_type=jnp.float32) + t2b[...])
        h = h * attn
    o_ref[...] = h


def _node_encode(x, ws, bs, tw1, tb1, tw2, tb2):
    xp = jnp.pad(x, ((0, NP_PAD - N), (0, 0)))
    gate_args = []
    for i in range(3):
        gate_args += [tw1[i].reshape(1, HID // 4), tb1[i].reshape(1, HID // 4),
                      tw2[i].reshape(HID // 4, 1), tb2[i].reshape(1, 1)]
    out = pl.pallas_call(
        _node_encode_body,
        grid=(NP_PAD // NODE_BLK,),
        in_specs=[pl.BlockSpec((NODE_BLK, F_IN), lambda i: (i, 0)),
                  _small(ws), _small(bs.reshape(1, HID))]
                 + [_small(a) for a in gate_args],
        out_specs=pl.BlockSpec((NODE_BLK, HID), lambda i: (i, 0)),
        out_shape=jax.ShapeDtypeStruct((NP_PAD, HID), jnp.float32),
    )(xp, ws, bs.reshape(1, HID), *gate_args)
    return out


def _edge_encode_body(ea_ref, ew1_ref, eb1_ref, ew2_ref, eb2_ref,
                      sw1_ref, sb1_ref, sw2_ref, sb2_ref, o_ref):
    ea = ea_ref[...]
    ew = jax.nn.sigmoid(
        jnp.dot(jax.nn.relu(jnp.dot(ea, ew1_ref[...], preferred_element_type=jnp.float32)
                            + eb1_ref[...]),
                ew2_ref[...], preferred_element_type=jnp.float32) + eb2_ref[...])
    sin_a = ea[:, D_EDGE - 2:D_EDGE - 1]
    cos_a = ea[:, D_EDGE - 1:D_EDGE]
    dist = 1.0 - ea[:, 0:1]
    sf = jnp.concatenate([sin_a, cos_a, dist], axis=1)
    se = jnp.tanh(
        jnp.dot(jax.nn.relu(jnp.dot(sf, sw1_ref[...], preferred_element_type=jnp.float32)
                            + sb1_ref[...]),
                sw2_ref[...], preferred_element_type=jnp.float32) + sb2_ref[...])
    o_ref[...] = ew * se


def _edge_encode(edge_attr, ew1, eb1, ew2, eb2, sw1, sb1, sw2, sb2):
    args = (ew1, eb1.reshape(1, HID), ew2, eb2.reshape(1, HID),
            sw1, sb1.reshape(1, HID // 2), sw2, sb2.reshape(1, HID))
    out = pl.pallas_call(
        _edge_encode_body,
        grid=(E // 4000,),
        in_specs=[pl.BlockSpec((4000, D_EDGE), lambda i: (i, 0))]
                 + [_small(a) for a in args],
        out_specs=pl.BlockSpec((4000, HID), lambda i: (i, 0)),
        out_shape=jax.ShapeDtypeStruct((E, HID), jnp.float32),
    )(edge_attr, *args)
    return out


def _mm_body(a_ref, w_ref, o_ref):
    o_ref[...] = jnp.dot(a_ref[...], w_ref[...], preferred_element_type=jnp.float32)


def _mm(a, w, blk):
    rows = a.shape[0]
    assert rows % blk == 0, (rows, blk)
    return pl.pallas_call(
        _mm_body,
        grid=(rows // blk,),
        in_specs=[pl.BlockSpec((blk, a.shape[1]), lambda i: (i, 0)), _small(w)],
        out_specs=pl.BlockSpec((blk, w.shape[1]), lambda i: (i, 0)),
        out_shape=jax.ShapeDtypeStruct((rows, w.shape[1]), jnp.float32),
    )(a, w)


def _mm2_body(a_ref, w1_ref, w2_ref, o1_ref, o2_ref):
    ab = a_ref[...]
    o1_ref[...] = jnp.dot(ab, w1_ref[...], preferred_element_type=jnp.float32)
    o2_ref[...] = jnp.dot(ab, w2_ref[...], preferred_element_type=jnp.float32)


def _mm2(a, w1, w2, blk):
    rows = a.shape[0]
    return pl.pallas_call(
        _mm2_body,
        grid=(rows // blk,),
        in_specs=[pl.BlockSpec((blk, a.shape[1]), lambda i: (i, 0)),
                  _small(w1), _small(w2)],
        out_specs=[pl.BlockSpec((blk, w1.shape[1]), lambda i: (i, 0)),
                   pl.BlockSpec((blk, w2.shape[1]), lambda i: (i, 0))],
        out_shape=[jax.ShapeDtypeStruct((rows, w1.shape[1]), jnp.float32),
                   jax.ShapeDtypeStruct((rows, w2.shape[1]), jnp.float32)],
    )(a, w1, w2)


def _logits_body(xls_ref, xrd_ref, ee_ref, attr_ref, o_ref):
    m = xls_ref[...] + xrd_ref[...] + ee_ref[...]
    m = jnp.maximum(m, 0.2 * m)
    p = m * attr_ref[...]
    cols = [jnp.sum(p[:, h * HID:(h + 1) * HID], axis=1, keepdims=True)
            for h in range(HEADS)]
    o_ref[...] = jnp.concatenate(cols, axis=1)


def _tc_logits(xls, xrd, ee, att):
    attr = att.reshape(1, HD)
    return pl.pallas_call(
        _logits_body,
        grid=(E2_PAD // EDGE_BLK,),
        in_specs=[pl.BlockSpec((EDGE_BLK, HD), lambda i: (i, 0)),
                  pl.BlockSpec((EDGE_BLK, HD), lambda i: (i, 0)),
                  pl.BlockSpec((EDGE_BLK, HD), lambda i: (i, 0)),
                  _small(attr)],
        out_specs=pl.BlockSpec((EDGE_BLK, HEADS), lambda i: (i, 0)),
        out_shape=jax.ShapeDtypeStruct((E2_PAD, HEADS), jnp.float32),
    )(xls, xrd, ee, attr)


def _scale_body(xls_ref, al_ref, o_ref):
    al = al_ref[...]
    x = xls_ref[...]
    parts = [x[:, h * HID:(h + 1) * HID] * al[:, h:h + 1] for h in range(HEADS)]
    o_ref[...] = jnp.concatenate(parts, axis=1)


def _tc_scale(xls, alpha):
    return pl.pallas_call(
        _scale_body,
        grid=(E2_PAD // EDGE_BLK,),
        in_specs=[pl.BlockSpec((EDGE_BLK, HD), lambda i: (i, 0)),
                  pl.BlockSpec((EDGE_BLK, HEADS), lambda i: (i, 0))],
        out_specs=pl.BlockSpec((EDGE_BLK, HD), lambda i: (i, 0)),
        out_shape=jax.ShapeDtypeStruct((E2_PAD, HD), jnp.float32),
    )(xls, alpha)


@functools.partial(jax.jit, static_argnums=())
def _sc_gather(table, idx):
    b_per_w = E2_PAD // NW
    nsteps = b_per_w // GCHUNK
    mesh = plsc.VectorSubcoreMesh(core_axis_name="c", subcore_axis_name="s")

    @functools.partial(
        pl.kernel, mesh=mesh,
        out_type=jax.ShapeDtypeStruct((E2_PAD, HD), jnp.float32),
        scratch_types=[
            pltpu.VMEM((b_per_w,), jnp.int32),
            pltpu.VMEM((GCHUNK, HD), jnp.float32),
            pltpu.SemaphoreType.DMA,
        ],
    )
    def k(table_hbm, idx_hbm, out_hbm, idx_v, rows_v, sem):
        wid = lax.axis_index("s") * 2 + lax.axis_index("c")
        base = wid * b_per_w
        pltpu.sync_copy(idx_hbm.at[pl.ds(base, b_per_w)], idx_v)

        @pl.loop(0, nsteps)
        def _(i):
            off = i * GCHUNK
            pltpu.async_copy(table_hbm.at[idx_v.at[pl.ds(off, GCHUNK)]],
                             rows_v, sem).wait()
            pltpu.sync_copy(rows_v, out_hbm.at[pl.ds(base + off, GCHUNK)])

    return k(table, idx)


def _gatv2_fast(h_pad, src2p, dst2p, ea2p, Wl, Wr, We, att, bias):
    xl2, xr2 = _mm2(h_pad, Wl, Wr, NODE_BLK)
    ee = _mm(ea2p, We, EDGE_BLK)
    xls = _sc_gather(xl2, src2p)
    xrd = _sc_gather(xr2, dst2p)
    logits = _tc_logits(xls, xrd, ee, att)
    lmax = jax.ops.segment_max(logits, dst2p, num_segments=N + 1)
    lmax = jnp.where(jnp.isfinite(lmax), lmax, 0.0)
    ex = jnp.exp(logits - lmax[dst2p])
    den = jax.ops.segment_sum(ex, dst2p, num_segments=N + 1)
    alpha = ex / (den[dst2p] + 1e-16)
    num_rows = _tc_scale(xls, alpha)
    out = jax.ops.segment_sum(num_rows, dst2p, num_segments=N + 1)[:N]
    out = out.reshape(N, HEADS, HID).mean(axis=1) + bias
    return out


def kernel(x, edge_index, edge_attr, ws, bs, tw1, tb1, tw2, tb2, sw1, sb1, sw2, sb2,
           ew1, eb1, ew2, eb2, Wl, Wr, We, att, gb, bn_g, bn_b,
           hw1, hb1, hw2, hb2, dw1, db1, dw2, db2):
    src, dst = edge_index[0], edge_index[1]
    h_pad = _node_encode(x, ws, bs, tw1, tb1, tw2, tb2)
    edge_weights = _edge_encode(edge_attr, ew1, eb1, ew2, eb2, sw1, sb1, sw2, sb2)

    ones = jnp.ones((E,), jnp.float32)
    cnt = jax.ops.segment_sum(ones, dst, num_segments=N)
    loop_attr = (jax.ops.segment_sum(edge_weights, dst, num_segments=N)
                 / jnp.maximum(cnt, 1.0)[:, None])

    ar = jnp.arange(N, dtype=src.dtype)
    npad = E2_PAD - E2
    src2p = jnp.concatenate([src, ar, jnp.zeros((npad,), src.dtype)])
    dst2p = jnp.concatenate([dst, ar, jnp.full((npad,), N, src.dtype)])
    ea2p = jnp.concatenate(
        [edge_weights, loop_attr, jnp.zeros((npad, HID), jnp.float32)], axis=0)

    h = jax.nn.elu(_gatv2_fast(h_pad, src2p, dst2p, ea2p,
                               Wl[0], Wr[0], We[0], att[0].reshape(HD), gb[0]))
    for l in range(1, 3):
        hp = jnp.pad(h, ((0, NP_PAD - N), (0, 0)))
        h_new = _gatv2_fast(hp, src2p, dst2p, ea2p,
                            Wl[l], Wr[l], We[l], att[l].reshape(HD), gb[l])
        h = jax.nn.elu(h + h_new)

    h = (h / jnp.sqrt(1.0 + 1e-5)) * bn_g + bn_b
    th = h[0:1]
    hc = jax.nn.relu(th @ hw1 + hb1) @ hw2 + hb2
    dp = jax.nn.relu(th @ dw1 + db1) @ dw2 + db2
    return jnp.concatenate([hc, dp], axis=1)


# speedup vs baseline: 4.0338x; 1.0046x over previous
import functools

import jax
import jax.numpy as jnp
from jax import lax
from jax.experimental import pallas as pl
from jax.experimental.pallas import tpu as pltpu
from jax.experimental.pallas import tpu_sc as plsc

N = 10000
E = 160000
F_IN = 128
D_EDGE = 16
HID = 64
HEADS = 4
HD = HEADS * HID

NP_PAD = 10240
NODE_BLK = 1024
E2 = E + N
E2_PAD = 172032
EDGE_BLK = 2048

NW = 32
GCHUNK = 192


def _small(a):
    return pl.BlockSpec(a.shape, lambda i: tuple(0 for _ in a.shape))


def _node_encode_body(x_ref, ws_ref, bs_ref,
                      t1w0, t1b0, t2w0, t2b0,
                      t1w1, t1b1, t2w1, t2b1,
                      t1w2, t1b2, t2w2, t2b2, o_ref):
    xb = x_ref[...]
    h = jnp.dot(xb, ws_ref[...], preferred_element_type=jnp.float32) + bs_ref[...]
    tf = xb[:, 0:1]
    scales = (1.0, 5.0, 20.0)
    gates = ((t1w0, t1b0, t2w0, t2b0), (t1w1, t1b1, t2w1, t2b1),
             (t1w2, t1b2, t2w2, t2b2))
    for i in range(3):
        t1w, t1b, t2w, t2b = gates[i]
        nt = tf / scales[i]
        a = jax.nn.relu(nt * t1w[...] + t1b[...])
        attn = jax.nn.sigmoid(
            jnp.dot(a, t2w[...], preferred_element_type=jnp.float32) + t2b[...])
        h = h * attn
    o_ref[...] = h


def _node_encode(x, ws, bs, tw1, tb1, tw2, tb2):
    xp = jnp.pad(x, ((0, NP_PAD - N), (0, 0)))
    gate_args = []
    for i in range(3):
        gate_args += [tw1[i].reshape(1, HID // 4), tb1[i].reshape(1, HID // 4),
                      tw2[i].reshape(HID // 4, 1), tb2[i].reshape(1, 1)]
    out = pl.pallas_call(
        _node_encode_body,
        grid=(NP_PAD // NODE_BLK,),
        in_specs=[pl.BlockSpec((NODE_BLK, F_IN), lambda i: (i, 0)),
                  _small(ws), _small(bs.reshape(1, HID))]
                 + [_small(a) for a in gate_args],
        out_specs=pl.BlockSpec((NODE_BLK, HID), lambda i: (i, 0)),
        out_shape=jax.ShapeDtypeStruct((NP_PAD, HID), jnp.float32),
    )(xp, ws, bs.reshape(1, HID), *gate_args)
    return out


def _edge_encode_body(ea_ref, ew1_ref, eb1_ref, ew2_ref, eb2_ref,
                      sw1_ref, sb1_ref, sw2_ref, sb2_ref, o_ref):
    ea = ea_ref[...]
    ew = jax.nn.sigmoid(
        jnp.dot(jax.nn.relu(jnp.dot(ea, ew1_ref[...], preferred_element_type=jnp.float32)
                            + eb1_ref[...]),
                ew2_ref[...], preferred_element_type=jnp.float32) + eb2_ref[...])
    sin_a = ea[:, D_EDGE - 2:D_EDGE - 1]
    cos_a = ea[:, D_EDGE - 1:D_EDGE]
    dist = 1.0 - ea[:, 0:1]
    sf = jnp.concatenate([sin_a, cos_a, dist], axis=1)
    se = jnp.tanh(
        jnp.dot(jax.nn.relu(jnp.dot(sf, sw1_ref[...], preferred_element_type=jnp.float32)
                            + sb1_ref[...]),
                sw2_ref[...], preferred_element_type=jnp.float32) + sb2_ref[...])
    o_ref[...] = ew * se


def _edge_encode(edge_attr, ew1, eb1, ew2, eb2, sw1, sb1, sw2, sb2):
    args = (ew1, eb1.reshape(1, HID), ew2, eb2.reshape(1, HID),
            sw1, sb1.reshape(1, HID // 2), sw2, sb2.reshape(1, HID))
    out = pl.pallas_call(
        _edge_encode_body,
        grid=(E // 4000,),
        in_specs=[pl.BlockSpec((4000, D_EDGE), lambda i: (i, 0))]
                 + [_small(a) for a in args],
        out_specs=pl.BlockSpec((4000, HID), lambda i: (i, 0)),
        out_shape=jax.ShapeDtypeStruct((E, HID), jnp.float32),
    )(edge_attr, *args)
    return out


def _mm_body(a_ref, w_ref, o_ref):
    o_ref[...] = jnp.dot(a_ref[...], w_ref[...], preferred_element_type=jnp.float32)


def _mm(a, w, blk):
    rows = a.shape[0]
    assert rows % blk == 0, (rows, blk)
    return pl.pallas_call(
        _mm_body,
        grid=(rows // blk,),
        in_specs=[pl.BlockSpec((blk, a.shape[1]), lambda i: (i, 0)), _small(w)],
        out_specs=pl.BlockSpec((blk, w.shape[1]), lambda i: (i, 0)),
        out_shape=jax.ShapeDtypeStruct((rows, w.shape[1]), jnp.float32),
    )(a, w)


def _mm2_body(a_ref, w1_ref, w2_ref, o1_ref, o2_ref):
    ab = a_ref[...]
    o1_ref[...] = jnp.dot(ab, w1_ref[...], preferred_element_type=jnp.float32)
    o2_ref[...] = jnp.dot(ab, w2_ref[...], preferred_element_type=jnp.float32)


def _mm2(a, w1, w2, blk):
    rows = a.shape[0]
    return pl.pallas_call(
        _mm2_body,
        grid=(rows // blk,),
        in_specs=[pl.BlockSpec((blk, a.shape[1]), lambda i: (i, 0)),
                  _small(w1), _small(w2)],
        out_specs=[pl.BlockSpec((blk, w1.shape[1]), lambda i: (i, 0)),
                   pl.BlockSpec((blk, w2.shape[1]), lambda i: (i, 0))],
        out_shape=[jax.ShapeDtypeStruct((rows, w1.shape[1]), jnp.float32),
                   jax.ShapeDtypeStruct((rows, w2.shape[1]), jnp.float32)],
    )(a, w1, w2)


def _logits_body(xls_ref, xrd_ref, ee_ref, attr_ref, o_ref):
    m = xls_ref[...] + xrd_ref[...] + ee_ref[...]
    m = jnp.maximum(m, 0.2 * m)
    p = m * attr_ref[...]
    cols = [jnp.sum(p[:, h * HID:(h + 1) * HID], axis=1, keepdims=True)
            for h in range(HEADS)]
    o_ref[...] = jnp.concatenate(cols, axis=1)


def _tc_logits(xls, xrd, ee, att):
    attr = att.reshape(1, HD)
    return pl.pallas_call(
        _logits_body,
        grid=(E2_PAD // EDGE_BLK,),
        in_specs=[pl.BlockSpec((EDGE_BLK, HD), lambda i: (i, 0)),
                  pl.BlockSpec((EDGE_BLK, HD), lambda i: (i, 0)),
                  pl.BlockSpec((EDGE_BLK, HD), lambda i: (i, 0)),
                  _small(attr)],
        out_specs=pl.BlockSpec((EDGE_BLK, HEADS), lambda i: (i, 0)),
        out_shape=jax.ShapeDtypeStruct((E2_PAD, HEADS), jnp.float32),
    )(xls, xrd, ee, attr)


def _scale_body(xls_ref, al_ref, o_ref):
    al = al_ref[...]
    x = xls_ref[...]
    parts = [x[:, h * HID:(h + 1) * HID] * al[:, h:h + 1] for h in range(HEADS)]
    o_ref[...] = jnp.concatenate(parts, axis=1)


def _tc_scale(xls, alpha):
    return pl.pallas_call(
        _scale_body,
        grid=(E2_PAD // EDGE_BLK,),
        in_specs=[pl.BlockSpec((EDGE_BLK, HD), lambda i: (i, 0)),
                  pl.BlockSpec((EDGE_BLK, HEADS), lambda i: (i, 0))],
        out_specs=pl.BlockSpec((EDGE_BLK, HD), lambda i: (i, 0)),
        out_shape=jax.ShapeDtypeStruct((E2_PAD, HD), jnp.float32),
    )(xls, alpha)


def _sc_gather2(tab_l, tab_r, idx_src, idx_dst):
    b_per_w = E2_PAD // NW
    nsteps = b_per_w // GCHUNK
    assert nsteps % 2 == 0
    mesh = plsc.VectorSubcoreMesh(core_axis_name="c", subcore_axis_name="s")

    @functools.partial(
        pl.kernel, mesh=mesh,
        out_type=[jax.ShapeDtypeStruct((E2_PAD, HD), jnp.float32),
                  jax.ShapeDtypeStruct((E2_PAD, HD), jnp.float32)],
        scratch_types=[
            pltpu.VMEM((b_per_w,), jnp.int32),
            pltpu.VMEM((GCHUNK, HD), jnp.float32),
            pltpu.VMEM((GCHUNK, HD), jnp.float32),
            pltpu.SemaphoreType.DMA,
            pltpu.SemaphoreType.DMA,
        ],
    )
    def k(tl_hbm, tr_hbm, is_hbm, id_hbm, ol_hbm, or_hbm,
          idx_v, buf0, buf1, sem0, sem1):
        wid = lax.axis_index("s") * 2 + lax.axis_index("c")
        base = wid * b_per_w

        def run_one(tab_hbm, i_hbm, o_hbm):
            pltpu.sync_copy(i_hbm.at[pl.ds(base, b_per_w)], idx_v)
            c0 = pltpu.async_copy(tab_hbm.at[idx_v.at[pl.ds(0, GCHUNK)]],
                                  buf0, sem0)

            @pl.loop(0, nsteps // 2)
            def _(j):
                off = 2 * j * GCHUNK
                c1 = pltpu.async_copy(
                    tab_hbm.at[idx_v.at[pl.ds(off + GCHUNK, GCHUNK)]],
                    buf1, sem1)
                pltpu.make_async_copy(tab_hbm.at[pl.ds(0, GCHUNK)], buf0,
                                      sem0).wait()
                pltpu.sync_copy(buf0, o_hbm.at[pl.ds(base + off, GCHUNK)])
                @pl.when(j + 1 < nsteps // 2)
                def _fire():
                    pltpu.async_copy(
                        tab_hbm.at[idx_v.at[pl.ds(off + 2 * GCHUNK, GCHUNK)]],
                        buf0, sem0)
                pltpu.make_async_copy(tab_hbm.at[pl.ds(0, GCHUNK)], buf1,
                                      sem1).wait()
                pltpu.sync_copy(buf1,
                                o_hbm.at[pl.ds(base + off + GCHUNK, GCHUNK)])

        run_one(tl_hbm, is_hbm, ol_hbm)
        run_one(tr_hbm, id_hbm, or_hbm)

    return k(tab_l, tab_r, idx_src, idx_dst)


def _gatv2_fast(h_pad, src2p, dst2p, ea2p, Wl, Wr, We, att, bias):
    xl2, xr2 = _mm2(h_pad, Wl, Wr, NODE_BLK)
    ee = _mm(ea2p, We, EDGE_BLK)
    xls, xrd = _sc_gather2(xl2, xr2, src2p, dst2p)
    logits = _tc_logits(xls, xrd, ee, att)
    lmax = jax.ops.segment_max(logits, dst2p, num_segments=N + 1)
    lmax = jnp.where(jnp.isfinite(lmax), lmax, 0.0)
    ex = jnp.exp(logits - lmax[dst2p])
    den = jax.ops.segment_sum(ex, dst2p, num_segments=N + 1)
    alpha = ex / (den[dst2p] + 1e-16)
    num_rows = _tc_scale(xls, alpha)
    out = jax.ops.segment_sum(num_rows, dst2p, num_segments=N + 1)[:N]
    out = out.reshape(N, HEADS, HID).mean(axis=1) + bias
    return out


def kernel(x, edge_index, edge_attr, ws, bs, tw1, tb1, tw2, tb2, sw1, sb1, sw2, sb2,
           ew1, eb1, ew2, eb2, Wl, Wr, We, att, gb, bn_g, bn_b,
           hw1, hb1, hw2, hb2, dw1, db1, dw2, db2):
    src, dst = edge_index[0], edge_index[1]
    h_pad = _node_encode(x, ws, bs, tw1, tb1, tw2, tb2)
    edge_weights = _edge_encode(edge_attr, ew1, eb1, ew2, eb2, sw1, sb1, sw2, sb2)

    ones = jnp.ones((E,), jnp.float32)
    cnt = jax.ops.segment_sum(ones, dst, num_segments=N)
    loop_attr = (jax.ops.segment_sum(edge_weights, dst, num_segments=N)
                 / jnp.maximum(cnt, 1.0)[:, None])

    ar = jnp.arange(N, dtype=src.dtype)
    npad = E2_PAD - E2
    src2p = jnp.concatenate([src, ar, jnp.zeros((npad,), src.dtype)])
    dst2p = jnp.concatenate([dst, ar, jnp.full((npad,), N, src.dtype)])
    ea2p = jnp.concatenate(
        [edge_weights, loop_attr, jnp.zeros((npad, HID), jnp.float32)], axis=0)

    h = jax.nn.elu(_gatv2_fast(h_pad, src2p, dst2p, ea2p,
                               Wl[0], Wr[0], We[0], att[0].reshape(HD), gb[0]))
    for l in range(1, 3):
        hp = jnp.pad(h, ((0, NP_PAD - N), (0, 0)))
        h_new = _gatv2_fast(hp, src2p, dst2p, ea2p,
                            Wl[l], Wr[l], We[l], att[l].reshape(HD), gb[l])
        h = jax.nn.elu(h + h_new)

    h = (h / jnp.sqrt(1.0 + 1e-5)) * bn_g + bn_b
    th = h[0:1]
    hc = jax.nn.relu(th @ hw1 + hb1) @ hw2 + hb2
    dp = jax.nn.relu(th @ dw1 + db1) @ dw2 + db2
    return jnp.concatenate([hc, dp], axis=1)
